# baseline (speedup 1.0000x reference)
import functools

import jax
import jax.numpy as jnp
from jax import lax
from jax.experimental import pallas as pl
from jax.experimental.pallas import tpu as pltpu
from jax.experimental.pallas import tpu_sc as plsc

N_FIELDS = 26
FIELD_DIM = 100000
N_FACTORS = 32
BATCH = 16384

NC, NS = 2, 16
NW = NC * NS
ROWS_PER_W = BATCH // NW
CHUNK = 64
N_CHUNKS = ROWS_PER_W // CHUNK
IDX_PER_CHUNK = CHUNK * N_FIELDS
GATHERS = IDX_PER_CHUNK // 128
X_PER_W = ROWS_PER_W * N_FIELDS


def _fm_body(x_hbm, emb_hbm, fc_hbm, w_hbm, b_hbm, out_hbm,
             xall, idx0, idx1, rows0, rows1, fcv0, fcv1, outv,
             wv_v, bv_v, sem0, sem1):
    cid = lax.axis_index("c")
    sid = lax.axis_index("s")
    wid = sid * NC + cid
    xbase = wid * X_PER_W

    pltpu.sync_copy(x_hbm.at[pl.ds(xbase, X_PER_W)], xall)
    pltpu.sync_copy(w_hbm, wv_v)
    pltpu.sync_copy(b_hbm, bv_v)
    Wv = wv_v[...]
    Bv = bv_v[...]

    iota16 = jnp.arange(16, dtype=jnp.int32)
    riota26 = iota16 * N_FIELDS

    idx_bufs = (idx0, idx1)
    rows_bufs = (rows0, rows1)
    fc_bufs = (fcv0, fcv1)
    sems = (sem0, sem1)

    def prep_fire(ci, p):
        idxP, rowsP, fcP, semP = idx_bufs[p], rows_bufs[p], fc_bufs[p], sems[p]

        def kbody(k, _):
            off = k * 16
            pos = off + iota16
            xv = xall[pl.ds(ci * IDX_PER_CHUNK + off, 16)]
            fld = lax.rem(pos, N_FIELDS)
            idxP[pl.ds(off, 16)] = xv + fld * FIELD_DIM
            return 0

        lax.fori_loop(0, IDX_PER_CHUNK // 16, kbody, 0)
        for j in range(GATHERS):
            isl = idxP.at[pl.ds(j * 128, 128)]
            pltpu.async_copy(emb_hbm.at[isl], rowsP.at[pl.ds(j * 128, 128)], semP)
            pltpu.async_copy(fc_hbm.at[isl], fcP.at[pl.ds(j * 128, 128)], semP)

    def drain(p):
        pltpu.make_async_copy(emb_hbm.at[pl.ds(0, IDX_PER_CHUNK)],
                              rows_bufs[p], sems[p]).wait()
        pltpu.make_async_copy(fc_hbm.at[pl.ds(0, IDX_PER_CHUNK)],
                              fc_bufs[p], sems[p]).wait()

    def compute(ci, p):
        rowsP, fcP = rows_bufs[p], fc_bufs[p]

        def gbody(g, _):
            rowbase = riota26 + g * (16 * N_FIELDS)

            def cbody(c, carry):
                Q, nrm = carry
                cvec = jnp.full((16,), c, dtype=jnp.int32)
                S = jnp.zeros((16,), jnp.float32)
                for f in range(N_FIELDS):
                    e = plsc.load_gather(rowsP, [rowbase + f, cvec])
                    S = S + e
                    Q = Q + e * e
                nrm = nrm + S * S
                return (Q, nrm)

            Q, nrm = lax.fori_loop(
                0, N_FACTORS, cbody,
                (jnp.zeros((16,), jnp.float32), jnp.zeros((16,), jnp.float32)))

            F = jnp.zeros((16,), jnp.float32)
            for f in range(N_FIELDS):
                F = F + plsc.load_gather(fcP, [rowbase + f])

            res = F * Wv + Bv + 0.5 * (nrm - Q)
            outv[pl.ds(ci * CHUNK + g * 16, 16)] = res
            return 0

        lax.fori_loop(0, CHUNK // 16, gbody, 0)

    prep_fire(0, 0)
    for ci in range(N_CHUNKS):
        p = ci & 1
        if ci + 1 < N_CHUNKS:
            prep_fire(ci + 1, 1 - p)
        drain(p)
        compute(ci, p)

    pltpu.sync_copy(outv, out_hbm.at[pl.ds(wid * ROWS_PER_W, ROWS_PER_W)])


_fm_sc = pl.kernel(
    _fm_body,
    out_type=jax.ShapeDtypeStruct((BATCH,), jnp.float32),
    mesh=plsc.VectorSubcoreMesh(core_axis_name="c", subcore_axis_name="s"),
    compiler_params=pltpu.CompilerParams(needs_layout_passes=False,
                                         use_tc_tiling_on_sc=False),
    scratch_types=[
        pltpu.VMEM((X_PER_W,), jnp.int32),
        pltpu.VMEM((IDX_PER_CHUNK,), jnp.int32),
        pltpu.VMEM((IDX_PER_CHUNK,), jnp.int32),
        pltpu.VMEM((IDX_PER_CHUNK, N_FACTORS), jnp.float32),
        pltpu.VMEM((IDX_PER_CHUNK, N_FACTORS), jnp.float32),
        pltpu.VMEM((IDX_PER_CHUNK,), jnp.float32),
        pltpu.VMEM((IDX_PER_CHUNK,), jnp.float32),
        pltpu.VMEM((ROWS_PER_W,), jnp.float32),
        pltpu.VMEM((16,), jnp.float32),
        pltpu.VMEM((16,), jnp.float32),
        pltpu.SemaphoreType.DMA,
        pltpu.SemaphoreType.DMA,
    ],
)


@jax.jit
def kernel(x, embedding, fc, W, b):
    x_flat = x.astype(jnp.int32).reshape(-1)
    fc_flat = fc.reshape(-1).astype(jnp.float32)
    wv = jnp.full((16,), W[0, 0], dtype=jnp.float32)
    bv = jnp.full((16,), b[0], dtype=jnp.float32)
    return _fm_sc(x_flat, embedding, fc_flat, wv, bv)
